# baseline (speedup 1.0000x reference)
import functools

import jax
import jax.numpy as jnp
from jax import lax
from jax.experimental import pallas as pl
from jax.experimental.pallas import tpu as pltpu
from jax.experimental.pallas import tpu_sc as plsc

D_MODEL = 1024
BATCH = 4
SEQ = 8192
TABLE_ROWS = 8195

NC = 2
NS = 16
NW = NC * NS
ROWS = BATCH * SEQ
ROWS_PER_W = ROWS // NW
W_PER_BATCH = SEQ // ROWS_PER_W
L = 16
R = L
T = ROWS_PER_W // R
VPB = ROWS_PER_W // L
NCOL = D_MODEL // L
NB = 3

_mesh = plsc.VectorSubcoreMesh(core_axis_name="c", subcore_axis_name="s")


@functools.partial(
    pl.kernel,
    mesh=_mesh,
    out_type=jax.ShapeDtypeStruct((ROWS, D_MODEL), jnp.float32),
    scratch_types=[
        pltpu.VMEM((SEQ,), jnp.int32),
        pltpu.VMEM((T, R), jnp.int32),
        pltpu.SMEM((T,), jnp.int32),
        pltpu.VMEM((R, D_MODEL), jnp.float32),
        pltpu.VMEM((R, D_MODEL), jnp.float32),
        pltpu.VMEM((R, D_MODEL), jnp.float32),
        pltpu.VMEM(((R + 1) * D_MODEL,), jnp.float32),
        pltpu.VMEM(((R + 1) * D_MODEL,), jnp.float32),
        pltpu.VMEM(((R + 1) * D_MODEL,), jnp.float32),
        pltpu.SemaphoreType.DMA,
        pltpu.SemaphoreType.DMA,
        pltpu.SemaphoreType.DMA,
        pltpu.SemaphoreType.DMA,
        pltpu.SemaphoreType.DMA,
        pltpu.SemaphoreType.DMA,
        pltpu.SemaphoreType.DMA,
        pltpu.SemaphoreType.DMA,
        pltpu.SemaphoreType.DMA,
    ],
    compiler_params=pltpu.CompilerParams(needs_layout_passes=False),
)
def _pos_emb_kernel(x_hbm, mask_hbm, table_hbm, out_hbm,
                    maskrow, rankbuf, starts,
                    xbuf0, xbuf1, xbuf2, tbuf0, tbuf1, tbuf2,
                    sx0, sx1, sx2, sg0, sg1, sg2, so0, so1, so2):
    wid = lax.axis_index("s") * NC + lax.axis_index("c")
    batch = wid // W_PER_BATCH
    sub = wid % W_PER_BATCH
    base = wid * ROWS_PER_W
    iota = lax.iota(jnp.int32, L)
    zero = jnp.zeros((L,), jnp.float32)

    xbufs = (xbuf0, xbuf1, xbuf2)
    tbufs = (tbuf0, tbuf1, tbuf2)
    sx = (sx0, sx1, sx2)
    sg = (sg0, sg1, sg2)
    so = (so0, so1, so2)

    for bb in range(NB):
        for c in range(NCOL):
            tbufs[bb][pl.ds(R * D_MODEL + c * L, L)] = zero

    pltpu.sync_copy(mask_hbm.at[batch], maskrow)

    def pre_body(i, acc):
        return acc + maskrow[pl.ds(i * L, L)]
    acc = lax.fori_loop(0, sub * VPB, pre_body,
                        jnp.zeros((L,), jnp.int32))
    carry0 = jnp.sum(acc)

    def ids_body(j, carry):
        v = maskrow[pl.ds((sub * VPB + j) * L, L)]
        cs = plsc.cumsum(v)
        ids = (cs + carry) * v
        start = jnp.minimum(carry + 1, TABLE_ROWS - R)
        starts[j] = start
        rankbuf[j, :] = jnp.where(v == 1, ids - start, R)
        return carry + jnp.sum(v)
    lax.fori_loop(0, VPB, ids_body, carry0)

    def issue_in(u, b):
        pltpu.async_copy(x_hbm.at[pl.ds(base + u * R, R)], xbufs[b], sx[b])
        pltpu.async_copy(
            table_hbm.at[pl.ds(starts[u] * D_MODEL, R * D_MODEL)],
            tbufs[b].at[pl.ds(0, R * D_MODEL)], sg[b])

    def wait_out(u, b):
        pltpu.make_async_copy(
            xbufs[b], out_hbm.at[pl.ds(base + u * R, R)], so[b]).wait()

    def step(u, b):
        bn = (b + 1) % NB
        @pl.when(u >= 2)
        def _():
            wait_out(u - 2, bn)

        @pl.when(u + 1 < T)
        def _():
            issue_in(u + 1, bn)

        pltpu.make_async_copy(x_hbm.at[pl.ds(base + u * R, R)],
                              xbufs[b], sx[b]).wait()
        pltpu.make_async_copy(table_hbm.at[pl.ds(0, R * D_MODEL)],
                              tbufs[b].at[pl.ds(0, R * D_MODEL)],
                              sg[b]).wait()

        tvec = jnp.full((L,), u, jnp.int32)

        @plsc.parallel_loop(0, R, unroll=1)
        def _row(r):
            rsp = plsc.load_gather(
                rankbuf, [tvec, jnp.full((L,), r, jnp.int32)])
            fbase = rsp * D_MODEL + iota
            for c in range(NCOL):
                v = plsc.load_gather(tbufs[b], [fbase + c * L])
                plsc.addupdate(xbufs[b].at[r, pl.ds(c * L, L)], v)

        pltpu.async_copy(xbufs[b], out_hbm.at[pl.ds(base + u * R, R)], so[b])

    issue_in(0, 0)

    def triple_body(i, _):
        for j in range(NB):
            step(i * NB + j, j)
        return 0
    lax.fori_loop(0, (T - 1) // NB, triple_body, 0)

    step(T - 1, (T - 1) % NB)
    wait_out(T - 2, (T - 2) % NB)
    wait_out(T - 1, (T - 1) % NB)


def kernel(x, mask, pos_table):
    x2 = x.reshape(ROWS, D_MODEL)
    out = _pos_emb_kernel(x2, mask, pos_table.reshape(-1))
    return out.reshape(BATCH, SEQ, D_MODEL)
